# baseline (speedup 1.0000x reference)
import functools

import jax
import jax.numpy as jnp
from jax import lax
from jax.experimental import pallas as pl
from jax.experimental.pallas import tpu as pltpu
from jax.experimental.pallas import tpu_sc as plsc

_LANES = 16
_C = 16


def _sc_dots(x3, yn3, WI, WO, seq_table, T, *, B, K, D, NC, NS):
    NW = NC * NS
    Bw = B // NW
    C = _C
    R = C * (K + 1)
    n_chunks = Bw // C
    JV = D // _LANES

    mesh = plsc.VectorSubcoreMesh(core_axis_name="c", subcore_axis_name="s")

    buf_set = [pltpu.VMEM((C, D), jnp.float32),
               pltpu.VMEM((R, D), jnp.float32),
               pltpu.VMEM((R, D), jnp.float32),
               pltpu.SemaphoreType.DMA]

    @functools.partial(
        pl.kernel,
        mesh=mesh,
        compiler_params=pltpu.CompilerParams(needs_layout_passes=False),
        out_type=jax.ShapeDtypeStruct((NW, _LANES), jnp.float32),
        scratch_types=[
            pltpu.VMEM((n_chunks, C), jnp.int32),
            pltpu.VMEM((n_chunks, R), jnp.int32),
            buf_set, buf_set, buf_set, buf_set,
            pltpu.VMEM((D,), jnp.float32),
            pltpu.VMEM((_LANES,), jnp.float32),
        ],
    )
    def run(x_h, yn_h, wi_h, wo_h, seq_h, t_h, part_h,
            ix2, iyn2, buf0, buf1, buf2, buf3, t_v, part_o):
        wid = lax.axis_index("s") * NC + lax.axis_index("c")
        pltpu.sync_copy(x_h.at[wid], ix2)
        pltpu.sync_copy(yn_h.at[wid], iyn2)
        pltpu.sync_copy(t_h, t_v)
        tj = [t_v[pl.ds(j * _LANES, _LANES)] for j in range(JV)]
        lane = lax.broadcasted_iota(jnp.int32, (_LANES,), 0)
        bufs = (buf0, buf1, buf2, buf3)
        NB = len(bufs)

        def gathers(c, buf):
            vi_b, wo_b, seq_b, sem = buf
            return [
                pltpu.make_async_copy(wi_h.at[ix2.at[c]], vi_b, sem),
                pltpu.make_async_copy(wo_h.at[iyn2.at[c]], wo_b, sem),
                pltpu.make_async_copy(seq_h.at[iyn2.at[c]], seq_b, sem),
            ]

        def issue(c, buf):
            for cp in gathers(c, buf):
                cp.start()

        def drain(c, buf):
            for cp in gathers(c, buf):
                cp.wait()

        def logsig(z):
            u = jnp.exp(-jnp.abs(z))
            big = u > (1.0 / 3.0)
            v = jnp.where(big, 0.5 * u - 0.5, u)
            y = jnp.float32(1.0 / 12.0)
            for m in range(11, 0, -1):
                y = jnp.float32(1.0 / m) - v * y
            y = v * y + jnp.where(big, jnp.float32(0.6931471805599453), 0.0)
            return jnp.minimum(z, 0.0) - y

        def compute(c, buf, part):
            vi_b, wo_b, seq_b, _ = buf

            def token_body(g, vecs):
                rn = C + g * K
                accs = [jnp.zeros((_LANES,), jnp.float32) for _ in range(K + 1)]
                for j in range(JV):
                    sl = pl.ds(j * _LANES, _LANES)
                    vij = vi_b[g, sl]
                    vitj = vij * tj[j]
                    accs[0] = (accs[0] + vij * wo_b[g, sl]
                               + vitj * seq_b[g, sl])
                    for k in range(K):
                        accs[k + 1] = (accs[k + 1] + vij * wo_b[rn + k, sl]
                                       + vitj * seq_b[rn + k, sl])
                m = lane == g
                return tuple(jnp.where(m, jnp.sum(accs[i]), vecs[i])
                             for i in range(K + 1))

            init = tuple(jnp.zeros((_LANES,), jnp.float32) for _ in range(K + 1))
            res = lax.fori_loop(0, C, token_body, init)
            p_acc, n_acc = part
            p_acc = p_acc + logsig(res[0])
            for k in range(K):
                n_acc = n_acc + logsig(-res[k + 1])
            return (p_acc, n_acc)

        for p in range(NB):
            issue(p, bufs[p])

        def ring_body(i, carry):
            for p in range(NB):
                c = NB * i + p
                drain(c, bufs[p])
                carry = compute(c, bufs[p], carry)
                nxt = c + NB

                @pl.when(nxt < n_chunks)
                def _():
                    issue(nxt, bufs[p])
            return carry

        main = n_chunks // NB
        zero = jnp.zeros((_LANES,), jnp.float32)
        part = lax.fori_loop(0, main, ring_body, (zero, zero))
        for c in range(NB * main, n_chunks):
            drain(c, bufs[c % NB])
            part = compute(c, bufs[c % NB], part)
        lane2 = lax.broadcasted_iota(jnp.int32, (_LANES,), 0)
        out_vec = jnp.where(lane2 == 0, jnp.sum(part[0]),
                            jnp.where(lane2 == 1, jnp.sum(part[1]), 0.0))
        part_o[...] = out_vec
        pltpu.sync_copy(part_o, part_h.at[wid])

    return run(x3, yn3, WI, WO, seq_table, T)


def kernel(x, y, neg_lookup, WI, WO, seq_table, T):
    B = x.shape[0]
    K = neg_lookup.shape[1]
    D = WI.shape[1]
    info = plsc.get_sparse_core_info()
    NC, NS = info.num_cores, info.num_subcores
    NW = NC * NS
    n_chunks = B // NW // _C
    yn = jnp.concatenate(
        [y.astype(jnp.int32).reshape(NW, n_chunks, _C),
         neg_lookup.astype(jnp.int32).reshape(NW, n_chunks, _C * K)], axis=2)
    parts = _sc_dots(
        x.astype(jnp.int32).reshape(NW, n_chunks, _C),
        yn, WI, WO, seq_table, T, B=B, K=K, D=D, NC=NC, NS=NS)
    return -(jnp.sum(parts[:, 0]) / B) - jnp.sum(parts[:, 1])


# speedup vs baseline: 1.0219x; 1.0219x over previous
import functools

import jax
import jax.numpy as jnp
from jax import lax
from jax.experimental import pallas as pl
from jax.experimental.pallas import tpu as pltpu
from jax.experimental.pallas import tpu_sc as plsc

_LANES = 16
_C = 16


def _sc_dots(x3, yn3, WI, WO, seq_table, T, *, B, K, D, NC, NS):
    NW = NC * NS
    Bw = B // NW
    C = _C
    R = C * (K + 1)
    n_chunks = Bw // C
    JV = D // _LANES

    mesh = plsc.VectorSubcoreMesh(core_axis_name="c", subcore_axis_name="s")

    buf_set = [pltpu.VMEM((C, D), jnp.float32),
               pltpu.VMEM((R, D), jnp.float32),
               pltpu.VMEM((R, D), jnp.float32),
               pltpu.SemaphoreType.DMA]

    @functools.partial(
        pl.kernel,
        mesh=mesh,
        compiler_params=pltpu.CompilerParams(needs_layout_passes=False),
        out_type=[
            jax.ShapeDtypeStruct((B,), jnp.float32),
            jax.ShapeDtypeStruct((NW, K, Bw), jnp.float32),
        ],
        scratch_types=[
            pltpu.VMEM((n_chunks, C), jnp.int32),
            pltpu.VMEM((n_chunks, R), jnp.int32),
            buf_set, buf_set, buf_set, buf_set,
            pltpu.VMEM((D,), jnp.float32),
            pltpu.VMEM((Bw,), jnp.float32),
            pltpu.VMEM((K, Bw), jnp.float32),
        ],
    )
    def run(x_h, yn_h, wi_h, wo_h, seq_h, t_h, pos_h, negd_h,
            ix2, iyn2, buf0, buf1, buf2, buf3, t_v, pos_o, neg_o):
        wid = lax.axis_index("s") * NC + lax.axis_index("c")
        base = wid * Bw
        pltpu.sync_copy(x_h.at[wid], ix2)
        pltpu.sync_copy(yn_h.at[wid], iyn2)
        pltpu.sync_copy(t_h, t_v)
        tj = [t_v[pl.ds(j * _LANES, _LANES)] for j in range(JV)]
        lane = lax.broadcasted_iota(jnp.int32, (_LANES,), 0)
        bufs = (buf0, buf1, buf2, buf3)
        NB = len(bufs)

        def gathers(c, buf):
            vi_b, wo_b, seq_b, sem = buf
            return [
                pltpu.make_async_copy(wi_h.at[ix2.at[c]], vi_b, sem),
                pltpu.make_async_copy(wo_h.at[iyn2.at[c]], wo_b, sem),
                pltpu.make_async_copy(seq_h.at[iyn2.at[c]], seq_b, sem),
            ]

        def issue(c, buf):
            for cp in gathers(c, buf):
                cp.start()

        def drain(c, buf):
            for cp in gathers(c, buf):
                cp.wait()

        def compute(c, buf):
            vi_b, wo_b, seq_b, _ = buf

            def token_body(g, vecs):
                rn = C + g * K
                accs = [jnp.zeros((_LANES,), jnp.float32) for _ in range(K + 1)]
                for j in range(JV):
                    sl = pl.ds(j * _LANES, _LANES)
                    vij = vi_b[g, sl]
                    vitj = vij * tj[j]
                    accs[0] = (accs[0] + vij * wo_b[g, sl]
                               + vitj * seq_b[g, sl])
                    for k in range(K):
                        accs[k + 1] = (accs[k + 1] + vij * wo_b[rn + k, sl]
                                       + vitj * seq_b[rn + k, sl])
                m = lane == g
                return tuple(jnp.where(m, jnp.sum(accs[i]), vecs[i])
                             for i in range(K + 1))

            init = tuple(jnp.zeros((_LANES,), jnp.float32) for _ in range(K + 1))
            res = lax.fori_loop(0, C, token_body, init)
            pos_o[pl.ds(c * C, C)] = res[0]
            for k in range(K):
                neg_o[k, pl.ds(c * C, C)] = res[k + 1]

        for p in range(NB):
            issue(p, bufs[p])

        def ring_body(i, carry):
            for p in range(NB):
                c = NB * i + p
                drain(c, bufs[p])
                compute(c, bufs[p])
                nxt = c + NB

                @pl.when(nxt < n_chunks)
                def _():
                    issue(nxt, bufs[p])
            return carry

        main = n_chunks // NB
        lax.fori_loop(0, main, ring_body, 0)
        for c in range(NB * main, n_chunks):
            drain(c, bufs[c % NB])
            compute(c, bufs[c % NB])
        pltpu.sync_copy(pos_o, pos_h.at[pl.ds(base, Bw)])
        pltpu.sync_copy(neg_o, negd_h.at[wid])

    return run(x3, yn3, WI, WO, seq_table, T)


def _tc_loss(pos2d, neg2d, B):
    def body(p_ref, n_ref, o_ref):
        p = p_ref[...]
        n = n_ref[...]
        ls_p = jnp.minimum(p, 0.0) - jnp.log(1.0 + jnp.exp(-jnp.abs(p)))
        ls_n = jnp.minimum(-n, 0.0) - jnp.log(1.0 + jnp.exp(-jnp.abs(n)))
        o_ref[0, 0] = -(jnp.sum(ls_p) / B) - jnp.sum(ls_n)

    return pl.pallas_call(
        body,
        out_shape=jax.ShapeDtypeStruct((1, 1), jnp.float32),
        out_specs=pl.BlockSpec(memory_space=pltpu.SMEM),
    )(pos2d, neg2d)


def kernel(x, y, neg_lookup, WI, WO, seq_table, T):
    B = x.shape[0]
    K = neg_lookup.shape[1]
    D = WI.shape[1]
    info = plsc.get_sparse_core_info()
    NC, NS = info.num_cores, info.num_subcores
    NW = NC * NS
    n_chunks = B // NW // _C
    yn = jnp.concatenate(
        [y.astype(jnp.int32).reshape(NW, n_chunks, _C),
         neg_lookup.astype(jnp.int32).reshape(NW, n_chunks, _C * K)], axis=2)
    pos, negd = _sc_dots(
        x.astype(jnp.int32).reshape(NW, n_chunks, _C),
        yn, WI, WO, seq_table, T, B=B, K=K, D=D, NC=NC, NS=NS)
    n_all = B * K
    loss = _tc_loss(pos.reshape(B // 128, 128), negd.reshape(n_all // 128, 128), B)
    return loss[0, 0]


# speedup vs baseline: 1.0420x; 1.0197x over previous
import functools

import jax
import jax.numpy as jnp
from jax import lax
from jax.experimental import pallas as pl
from jax.experimental.pallas import tpu as pltpu
from jax.experimental.pallas import tpu_sc as plsc

_LANES = 16
_C = 16


def _sc_dots(x3, yn3, WI, WO, seq_table, T, *, B, K, D, NC, NS):
    NW = NC * NS
    Bw = B // NW
    C = _C
    R = C * (K + 1)
    n_chunks = Bw // C
    JV = D // _LANES

    mesh = plsc.VectorSubcoreMesh(core_axis_name="c", subcore_axis_name="s")

    buf_set = [pltpu.VMEM((C, D), jnp.float32),
               pltpu.VMEM((R, D), jnp.float32),
               pltpu.VMEM((R, D), jnp.float32),
               pltpu.SemaphoreType.DMA]

    @functools.partial(
        pl.kernel,
        mesh=mesh,
        compiler_params=pltpu.CompilerParams(needs_layout_passes=False),
        out_type=[
            jax.ShapeDtypeStruct((B,), jnp.float32),
            jax.ShapeDtypeStruct((NW, K, Bw), jnp.float32),
        ],
        scratch_types=[
            pltpu.VMEM((n_chunks, C), jnp.int32),
            pltpu.VMEM((n_chunks, R), jnp.int32),
            buf_set, buf_set, buf_set,
            pltpu.VMEM((D,), jnp.float32),
            pltpu.VMEM((Bw,), jnp.float32),
            pltpu.VMEM((K, Bw), jnp.float32),
        ],
    )
    def run(x_h, yn_h, wi_h, wo_h, seq_h, t_h, pos_h, negd_h,
            ix2, iyn2, buf0, buf1, buf2, t_v, pos_o, neg_o):
        wid = lax.axis_index("s") * NC + lax.axis_index("c")
        base = wid * Bw
        pltpu.sync_copy(x_h.at[wid], ix2)
        pltpu.sync_copy(yn_h.at[wid], iyn2)
        pltpu.sync_copy(t_h, t_v)
        tj = [t_v[pl.ds(j * _LANES, _LANES)] for j in range(JV)]
        lane = lax.broadcasted_iota(jnp.int32, (_LANES,), 0)
        bufs = (buf0, buf1, buf2)
        NB = len(bufs)

        def gathers(c, buf):
            vi_b, wo_b, seq_b, sem = buf
            return [
                pltpu.make_async_copy(wi_h.at[ix2.at[c]], vi_b, sem),
                pltpu.make_async_copy(wo_h.at[iyn2.at[c]], wo_b, sem),
                pltpu.make_async_copy(seq_h.at[iyn2.at[c]], seq_b, sem),
            ]

        def issue(c, buf):
            for cp in gathers(c, buf):
                cp.start()

        def drain(c, buf):
            for cp in gathers(c, buf):
                cp.wait()

        def compute(c, buf):
            vi_b, wo_b, seq_b, _ = buf

            def token_body(g, vecs):
                r0 = g * (K + 1)
                accs = [jnp.zeros((_LANES,), jnp.float32) for _ in range(K + 1)]
                for j in range(JV):
                    sl = pl.ds(j * _LANES, _LANES)
                    vij = vi_b[g, sl]
                    vitj = vij * tj[j]
                    for k in range(K + 1):
                        accs[k] = (accs[k] + vij * wo_b[r0 + k, sl]
                                   + vitj * seq_b[r0 + k, sl])
                m = lane == g
                return tuple(jnp.where(m, jnp.sum(accs[i]), vecs[i])
                             for i in range(K + 1))

            init = tuple(jnp.zeros((_LANES,), jnp.float32) for _ in range(K + 1))
            res = lax.fori_loop(0, C, token_body, init)
            pos_o[pl.ds(c * C, C)] = res[0]
            for k in range(K):
                neg_o[k, pl.ds(c * C, C)] = res[k + 1]

        for p in range(NB):
            issue(p, bufs[p])

        def ring_body(i, carry):
            for p in range(NB):
                c = NB * i + p
                drain(c, bufs[p])
                compute(c, bufs[p])
                nxt = c + NB

                @pl.when(nxt < n_chunks)
                def _():
                    issue(nxt, bufs[p])
            return carry

        main = n_chunks // NB
        lax.fori_loop(0, main, ring_body, 0)
        for c in range(NB * main, n_chunks):
            drain(c, bufs[c % NB])
            compute(c, bufs[c % NB])
        pltpu.sync_copy(pos_o, pos_h.at[pl.ds(base, Bw)])
        pltpu.sync_copy(neg_o, negd_h.at[wid])

    return run(x3, yn3, WI, WO, seq_table, T)


def _tc_loss(pos2d, neg2d, B):
    def body(p_ref, n_ref, o_ref):
        p = p_ref[...]
        n = n_ref[...]
        ls_p = jnp.minimum(p, 0.0) - jnp.log(1.0 + jnp.exp(-jnp.abs(p)))
        ls_n = jnp.minimum(-n, 0.0) - jnp.log(1.0 + jnp.exp(-jnp.abs(n)))
        o_ref[0, 0] = -(jnp.sum(ls_p) / B) - jnp.sum(ls_n)

    return pl.pallas_call(
        body,
        out_shape=jax.ShapeDtypeStruct((1, 1), jnp.float32),
        out_specs=pl.BlockSpec(memory_space=pltpu.SMEM),
    )(pos2d, neg2d)


def kernel(x, y, neg_lookup, WI, WO, seq_table, T):
    B = x.shape[0]
    K = neg_lookup.shape[1]
    D = WI.shape[1]
    info = plsc.get_sparse_core_info()
    NC, NS = info.num_cores, info.num_subcores
    NW = NC * NS
    n_chunks = B // NW // _C
    yn = jnp.concatenate(
        [y.astype(jnp.int32)[:, None], neg_lookup.astype(jnp.int32)], axis=1)
    yn = yn.reshape(NW, n_chunks, _C * (K + 1))
    pos, negd = _sc_dots(
        x.astype(jnp.int32).reshape(NW, n_chunks, _C),
        yn, WI, WO, seq_table, T, B=B, K=K, D=D, NC=NC, NS=NS)
    n_all = B * K
    loss = _tc_loss(pos.reshape(B // 128, 128), negd.reshape(n_all // 128, 128), B)
    return loss[0, 0]
